# baseline (speedup 1.0000x reference)
import functools

import jax
import jax.numpy as jnp
from jax import lax
from jax.experimental import pallas as pl
from jax.experimental.pallas import tpu as pltpu
from jax.experimental.pallas import tpu_sc as plsc

_C = 80
_NBUF = 4
_W = 2


def kernel(x, u):
    n, d = x.shape
    nw = 32
    assert n % _C == 0
    nchunk = n // _C
    kmax = -(-nchunk // nw)

    mesh = plsc.VectorSubcoreMesh(core_axis_name="c", subcore_axis_name="s")

    @functools.partial(
        pl.kernel,
        out_type=jax.ShapeDtypeStruct((n, d), x.dtype),
        mesh=mesh,
        scratch_types=[
            pltpu.VMEM((_NBUF, _C, d), jnp.float32),
            pltpu.SemaphoreType.DMA((_NBUF,)),
            pltpu.SemaphoreType.DMA((_NBUF,)),
        ],
    )
    def _copy(x_hbm, o_hbm, bufs, in_sems, out_sems):
        wid = lax.axis_index("s") * mesh.num_cores + lax.axis_index("c")

        def in_cp(k):
            j = wid + k * nw
            b = k % _NBUF
            return pltpu.make_async_copy(
                x_hbm.at[pl.ds(j * _C, _C), :], bufs.at[b], in_sems.at[b])

        def out_cp(k):
            j = wid + k * nw
            b = k % _NBUF
            return pltpu.make_async_copy(
                bufs.at[b], o_hbm.at[pl.ds(j * _C, _C), :], out_sems.at[b])

        def valid(k):
            return wid + k * nw < nchunk

        for t in range(min(_W, kmax)):
            @pl.when(valid(t))
            def _(t=t):
                in_cp(t).start()

        for k in range(kmax):
            nxt = k + _W
            if nxt < kmax:
                @pl.when(valid(nxt))
                def _(k=k, nxt=nxt):
                    prev = nxt - _NBUF
                    if prev >= 0:
                        out_cp(prev).wait()
                    in_cp(nxt).start()

            @pl.when(valid(k))
            def _(k=k):
                in_cp(k).wait()
                out_cp(k).start()

        for j in range(kmax):
            if j + _NBUF < kmax:
                cond = valid(j) & jnp.logical_not(valid(j + _NBUF))
            else:
                cond = valid(j)

            @pl.when(cond)
            def _(j=j):
                out_cp(j).wait()

    return _copy(x)


# speedup vs baseline: 1.3367x; 1.3367x over previous
import jax
import jax.numpy as jnp
from jax.experimental import pallas as pl
from jax.experimental.pallas import tpu as pltpu

_NBUF = 8
_W = 4


def _schedule(n):
    small, nsmall = 400, 4
    if n <= 2 * small * nsmall:
        c = max(8, n // 16 // 8 * 8)
        sizes = [c] * (n // c)
        if n % c:
            sizes.append(n % c)
        return sizes
    mid = n - 2 * small * nsmall
    nbig = max(1, mid // 4400)
    big = mid // nbig // 8 * 8
    sizes = [small] * nsmall + [big] * nbig + [small] * nsmall
    rem = n - sum(sizes)
    assert rem >= 0 and rem % 8 == 0
    if rem:
        sizes.insert(nsmall, rem)
    return sizes


def _make_relay(sizes, d):
    offs = [0]
    for s in sizes:
        offs.append(offs[-1] + s)
    nchunk = len(sizes)
    bufrows = max(sizes)

    def _relay(x_hbm, o_hbm, bufs, in_sems, out_sems):
        def in_cp(i):
            b = i % _NBUF
            return pltpu.make_async_copy(
                x_hbm.at[pl.ds(offs[i], sizes[i]), :],
                bufs.at[b, pl.ds(0, sizes[i]), :],
                in_sems.at[b])

        def out_cp(i):
            b = i % _NBUF
            return pltpu.make_async_copy(
                bufs.at[b, pl.ds(0, sizes[i]), :],
                o_hbm.at[pl.ds(offs[i], sizes[i]), :],
                out_sems.at[b])

        for i in range(min(_W, nchunk)):
            in_cp(i).start()
        waited_out = 0
        for i in range(nchunk):
            nxt = i + _W
            if nxt < nchunk:
                prev = nxt - _NBUF
                if prev >= 0:
                    out_cp(prev).wait()
                    waited_out = prev + 1
                in_cp(nxt).start()
            in_cp(i).wait()
            out_cp(i).start()
        for i in range(waited_out, nchunk):
            out_cp(i).wait()

    return _relay, nchunk, bufrows


def kernel(x, u):
    n, d = x.shape
    sizes = _schedule(n)
    relay, nchunk, bufrows = _make_relay(sizes, d)
    return pl.pallas_call(
        relay,
        in_specs=[pl.BlockSpec(memory_space=pl.ANY)],
        out_specs=pl.BlockSpec(memory_space=pl.ANY),
        out_shape=jax.ShapeDtypeStruct((n, d), x.dtype),
        scratch_shapes=[
            pltpu.VMEM((_NBUF, bufrows, d), jnp.float32),
            pltpu.SemaphoreType.DMA((_NBUF,)),
            pltpu.SemaphoreType.DMA((_NBUF,)),
        ],
    )(x)
